# baseline (speedup 1.0000x reference)
import functools

import jax
import jax.numpy as jnp
from jax.experimental import pallas as pl
from jax.experimental.pallas import tpu as pltpu

HID = 64
KCAP = 8
KMIN = 2
KMAX = 8
NITER = 3
SEQ = 50


def _softmax(b):
    m = jnp.max(b, axis=-1, keepdims=True)
    e = jnp.exp(b - m)
    return e / jnp.sum(e, axis=-1, keepdims=True)


def _squash(caps):
    cn = jnp.sum(caps * caps, axis=-1, keepdims=True)
    return caps * (cn / (1.0 + cn) / jnp.sqrt(cn + 1e-9))


def _ckl_blh(c, mat):
    return jax.lax.dot_general(
        c, mat, (((2,), (1,)), ((0,), (0,))),
        preferred_element_type=jnp.float32,
        precision=jax.lax.Precision.HIGHEST,
    )


def _blh_bkh(mat, caps):
    return jax.lax.dot_general(
        caps, mat, (((2,), (2,)), ((0,), (0,))),
        preferred_element_type=jnp.float32,
        precision=jax.lax.Precision.HIGHEST,
    )


def _block_kernel(x_ref, w_ref, b0_ref, opt_ref, scores_ref, *, bb):
    x2 = x_ref[...]
    x = x2.reshape(bb, SEQ, HID)
    w = w_ref[...]
    lin = jax.lax.dot_general(
        x2, w, (((1,), (1,)), ((), ())),
        preferred_element_type=jnp.float32,
    ).reshape(bb, SEQ, HID)

    b = b0_ref[...]
    for t in range(NITER - 1):
        m = jnp.max(b, axis=-1, keepdims=True)
        e = jnp.exp(b - m)
        inv = 1.0 / jnp.sum(e, axis=-1)
        caps = _squash(_ckl_blh(e, lin) * inv[:, :, None])
        b = b + _blh_bkh(lin, caps)

    c = _softmax(b)
    caps_u = _ckl_blh(c, lin)
    cn = jnp.sum(caps_u * caps_u, axis=-1)
    strength = cn / (1.0 + cn) / jnp.sqrt(cn + 1e-9) * jnp.sqrt(cn)

    e_nrm = jnp.sqrt(jnp.sum(x * x, axis=-1, keepdims=True))
    e_norm = x / (e_nrm + 1e-8)

    csum = jnp.sum(c, axis=-1)
    mu = jax.lax.dot_general(
        c, x, (((2,), (1,)), ((0,), (0,))),
        preferred_element_type=jnp.float32,
    ) / (csum[:, :, None] + 1e-8)
    mu_nrm = jnp.sqrt(jnp.sum(mu * mu, axis=-1, keepdims=True))
    mu_norm = mu / (mu_nrm + 1e-8)

    cos = jax.lax.dot_general(
        mu_norm, e_norm, (((2,), (2,)), ((0,), (0,))),
        preferred_element_type=jnp.float32,
    )
    s = jnp.sum(c * cos, axis=-1)
    pair = jax.lax.dot_general(
        mu_norm, mu_norm, (((2,), (2,)), ((0,), (0,))),
        preferred_element_type=jnp.float32,
    )

    si = strength[:, :, None]
    sj = strength[:, None, :]
    ii = jax.lax.broadcasted_iota(jnp.int32, (KCAP, KCAP), 0)
    jj = jax.lax.broadcasted_iota(jnp.int32, (KCAP, KCAP), 1)
    beats = (si > sj) | ((si == sj) & (ii < jj)[None, :, :])
    rank = jnp.sum(beats.astype(jnp.int32), axis=1)
    offdiag = (ii != jj).astype(jnp.float32)[None, :, :]

    w = pair * offdiag
    mxr = jnp.maximum(rank[:, :, None], rank[:, None, :])
    score_cols = []
    for k in range(KMIN, KMAX + 1):
        cons = jnp.sum(jnp.where(rank < k, s, 0.0), axis=-1) / float(k * SEQ)
        psum = 0.5 * jnp.sum(jnp.where(mxr < k, w, 0.0), axis=(1, 2))
        div = 1.0 - (2.0 / float(k * (k - 1))) * psum
        score_cols.append(0.5 * cons + 0.5 * div)
    scores = jnp.stack(score_cols, axis=1)

    opt = jnp.argmax(scores, axis=1).astype(jnp.int32) + KMIN
    scores_ref[...] = scores
    opt_ref[...] = opt[:, None]


_B0_CACHE = {}


def _b0_const(bsz, seq):
    shape = (bsz, KCAP, seq)
    if shape not in _B0_CACHE:
        with jax.ensure_compile_time_eval():
            _B0_CACHE[shape] = jax.random.normal(
                jax.random.key(1), shape, dtype=jnp.float32
            )
    return _B0_CACHE[shape]


def kernel(user_features, W):
    bsz, seq, hid = user_features.shape
    bb = 64
    grid = (bsz // bb,)
    b0 = _b0_const(bsz, seq)


    opt, scores = pl.pallas_call(
        functools.partial(_block_kernel, bb=bb),
        grid=grid,
        in_specs=[
            pl.BlockSpec((bb * seq, hid), lambda i: (i, 0)),
            pl.BlockSpec((hid, hid), lambda i: (0, 0)),
            pl.BlockSpec((bb, KCAP, seq), lambda i: (i, 0, 0)),
        ],
        out_specs=[
            pl.BlockSpec((bb, 1), lambda i: (i, 0)),
            pl.BlockSpec((bb, KMAX - KMIN + 1), lambda i: (i, 0)),
        ],
        out_shape=[
            jax.ShapeDtypeStruct((bsz, 1), jnp.int32),
            jax.ShapeDtypeStruct((bsz, KMAX - KMIN + 1), jnp.float32),
        ],
        compiler_params=pltpu.CompilerParams(
            dimension_semantics=("parallel",),
        ),
    )(user_features.reshape(bsz * seq, hid), W, b0)

    return (opt.reshape(bsz), scores)


# speedup vs baseline: 1.1779x; 1.1779x over previous
import functools

import jax
import jax.numpy as jnp
from jax.experimental import pallas as pl
from jax.experimental.pallas import tpu as pltpu

HID = 64
KCAP = 8
KMIN = 2
KMAX = 8
NITER = 3
SEQ = 50


def _softmax(b):
    m = jnp.max(b, axis=-1, keepdims=True)
    e = jnp.exp(b - m)
    return e / jnp.sum(e, axis=-1, keepdims=True)


def _squash(caps):
    cn = jnp.sum(caps * caps, axis=-1, keepdims=True)
    return caps * (cn / (1.0 + cn) / jnp.sqrt(cn + 1e-9))


def _ckl_blh(c, mat):
    return jax.lax.dot_general(
        c, mat, (((2,), (1,)), ((0,), (0,))),
        preferred_element_type=jnp.float32,
        precision=jax.lax.Precision.HIGHEST,
    )


def _blh_bkh(mat, caps):
    return jax.lax.dot_general(
        caps, mat, (((2,), (2,)), ((0,), (0,))),
        preferred_element_type=jnp.float32,
        precision=jax.lax.Precision.HIGHEST,
    )


def _block_kernel(x_ref, w_ref, b0_ref, c0_ref, opt_ref, scores_ref, *, bb):
    x = x_ref[...]
    w = w_ref[...]
    lin = jax.lax.dot_general(
        x.reshape(bb * SEQ, HID), w, (((1,), (1,)), ((), ())),
        preferred_element_type=jnp.float32,
    ).reshape(bb, SEQ, HID)

    caps = _squash(_ckl_blh(c0_ref[...], lin))
    b = b0_ref[...] + _blh_bkh(lin, caps)
    for t in range(1, NITER - 1):
        m = jnp.max(b, axis=-1, keepdims=True)
        e = jnp.exp(b - m)
        inv = 1.0 / jnp.sum(e, axis=-1)
        caps = _squash(_ckl_blh(e, lin) * inv[:, :, None])
        b = b + _blh_bkh(lin, caps)

    c = _softmax(b)
    caps_u = _ckl_blh(c, lin)
    cn = jnp.sum(caps_u * caps_u, axis=-1)
    strength = cn / (1.0 + cn) / jnp.sqrt(cn + 1e-9) * jnp.sqrt(cn)

    e_nrm = jnp.sqrt(jnp.sum(x * x, axis=-1, keepdims=True))
    e_norm = x / (e_nrm + 1e-8)

    csum = jnp.sum(c, axis=-1)
    mu = jax.lax.dot_general(
        c, x, (((2,), (1,)), ((0,), (0,))),
        preferred_element_type=jnp.float32,
    ) / (csum[:, :, None] + 1e-8)
    mu_nrm = jnp.sqrt(jnp.sum(mu * mu, axis=-1, keepdims=True))
    mu_norm = mu / (mu_nrm + 1e-8)

    cos = jax.lax.dot_general(
        mu_norm, e_norm, (((2,), (2,)), ((0,), (0,))),
        preferred_element_type=jnp.float32,
    )
    s = jnp.sum(c * cos, axis=-1)
    pair = jax.lax.dot_general(
        mu_norm, mu_norm, (((2,), (2,)), ((0,), (0,))),
        preferred_element_type=jnp.float32,
    )

    si = strength[:, :, None]
    sj = strength[:, None, :]
    ii = jax.lax.broadcasted_iota(jnp.int32, (KCAP, KCAP), 0)
    jj = jax.lax.broadcasted_iota(jnp.int32, (KCAP, KCAP), 1)
    beats = (si > sj) | ((si == sj) & (ii < jj)[None, :, :])
    rank = jnp.sum(beats.astype(jnp.int32), axis=1)
    offdiag = (ii != jj).astype(jnp.float32)[None, :, :]

    w = pair * offdiag
    mxr = jnp.maximum(rank[:, :, None], rank[:, None, :])
    score_cols = []
    for k in range(KMIN, KMAX + 1):
        cons = jnp.sum(jnp.where(rank < k, s, 0.0), axis=-1) / float(k * SEQ)
        psum = 0.5 * jnp.sum(jnp.where(mxr < k, w, 0.0), axis=(1, 2))
        div = 1.0 - (2.0 / float(k * (k - 1))) * psum
        score_cols.append(0.5 * cons + 0.5 * div)
    scores = jnp.stack(score_cols, axis=1)

    opt = jnp.argmax(scores, axis=1).astype(jnp.int32) + KMIN
    scores_ref[...] = scores
    opt_ref[...] = opt[:, None]


_B0_CACHE = {}


def _b0_const(bsz, seq):
    shape = (bsz, KCAP, seq)
    if shape not in _B0_CACHE:
        with jax.ensure_compile_time_eval():
            b0 = jax.random.normal(jax.random.key(1), shape, dtype=jnp.float32)
            _B0_CACHE[shape] = (b0, jax.nn.softmax(b0, axis=-1))
    return _B0_CACHE[shape]


def kernel(user_features, W):
    bsz, seq, hid = user_features.shape
    bb = 64
    grid = (bsz // bb,)
    b0, c0 = _b0_const(bsz, seq)


    opt, scores = pl.pallas_call(
        functools.partial(_block_kernel, bb=bb),
        grid=grid,
        in_specs=[
            pl.BlockSpec((bb, seq, hid), lambda i: (i, 0, 0)),
            pl.BlockSpec((hid, hid), lambda i: (0, 0)),
            pl.BlockSpec((bb, KCAP, seq), lambda i: (i, 0, 0)),
            pl.BlockSpec((bb, KCAP, seq), lambda i: (i, 0, 0)),
        ],
        out_specs=[
            pl.BlockSpec((bb, 1), lambda i: (i, 0)),
            pl.BlockSpec((bb, KMAX - KMIN + 1), lambda i: (i, 0)),
        ],
        out_shape=[
            jax.ShapeDtypeStruct((bsz, 1), jnp.int32),
            jax.ShapeDtypeStruct((bsz, KMAX - KMIN + 1), jnp.float32),
        ],
        compiler_params=pltpu.CompilerParams(
            dimension_semantics=("parallel",),
        ),
    )(user_features, W, b0, c0)

    return (opt.reshape(bsz), scores)
